# baseline (speedup 1.0000x reference)
import functools

import jax
import jax.numpy as jnp
from jax import lax
from jax.experimental import pallas as pl
from jax.experimental.pallas import tpu as pltpu
from jax.experimental.pallas import tpu_sc as plsc

NC = 2
NS = 16
NW = NC * NS
LANES = 16
CHUNK = 128
GCHUNKS = 4
BK = CHUNK * GCHUNKS


def _sc_mesh(num_cores=NC):
    return plsc.VectorSubcoreMesh(core_axis_name="c", subcore_axis_name="s",
                                  num_cores=num_cores, num_subcores=NS)


def _gather_call(nodes, s2d, r2d, E, D):
    n_groups = E // BK
    steps = (n_groups + NW - 1) // NW
    dt = nodes.dtype

    @functools.partial(
        pl.kernel,
        out_type=(jax.ShapeDtypeStruct((E, D), dt),
                  jax.ShapeDtypeStruct((E, D), dt)),
        mesh=_sc_mesh(),
        scratch_types=[
            pltpu.VMEM((GCHUNKS, CHUNK), jnp.int32),
            pltpu.VMEM((BK, D), dt),
            pltpu.SemaphoreType.DMA,
        ],
        compiler_params=pltpu.CompilerParams(use_tc_tiling_on_sc=False),
    )
    def k(nodes_h, s_h, r_h, sf_h, rf_h, idx_v, rows_v, sem):
        wid = lax.axis_index("s") * NC + lax.axis_index("c")

        @pl.loop(0, steps)
        def _(i):
            b = wid + i * NW

            @pl.when(b < n_groups)
            def _():
                ebase = b * BK
                rbase = b * GCHUNKS

                def do(idx_src, dst_h):
                    pltpu.sync_copy(idx_src.at[pl.ds(rbase, GCHUNKS)], idx_v)
                    cps = [
                        pltpu.async_copy(
                            nodes_h.at[idx_v.at[j]],
                            rows_v.at[pl.ds(j * CHUNK, CHUNK)],
                            sem,
                        )
                        for j in range(GCHUNKS)
                    ]
                    for c in cps:
                        c.wait()
                    pltpu.sync_copy(rows_v, dst_h.at[pl.ds(ebase, BK)])

                do(s_h, sf_h)
                do(r_h, rf_h)

    return k(nodes, s2d, r2d)


NPH = 5120
DUMP = 512
ACC_R = NPH + DUMP


def _remap_call(r_rs):
    R0, R1 = r_rs.shape

    def body(r_ref, o_ref):
        idx = r_ref[...]
        for c in range(NC):
            rel = idx - c * NPH
            inr = (rel >= 0) & (rel < NPH)
            o_ref[c, :, :] = jnp.where(inr, rel, NPH + (idx & (DUMP - 1)))

    return pl.pallas_call(
        body,
        out_shape=jax.ShapeDtypeStruct((NC, R0, R1), jnp.int32),
    )(r_rs)


def _scatter_call(msgs, idx_rm, E, D):
    n_groups = E // BK
    rows_per_tile = ACC_R // NS
    zrows = 32
    assert rows_per_tile % zrows == 0

    @functools.partial(
        pl.kernel,
        out_type=jax.ShapeDtypeStruct((NC, ACC_R, D), jnp.float32),
        mesh=_sc_mesh(),
        scratch_types=[
            pltpu.VMEM((GCHUNKS, CHUNK), jnp.int32),
            pltpu.VMEM((BK, D), jnp.float32),
            pltpu.VMEM((zrows, D), jnp.float32),
            pltpu.VMEM_SHARED((ACC_R, D), jnp.float32),
        ],
    )
    def k(m_h, r_h, out_h, idx_v, vals_v, zbuf_v, acc_s):
        cid = lax.axis_index("c")
        sid = lax.axis_index("s")

        @pl.loop(0, zrows)
        def _(r):
            @pl.loop(0, D, step=LANES)
            def _(c2):
                zbuf_v[r, pl.ds(c2, LANES)] = jnp.zeros((LANES,), jnp.float32)

        @pl.loop(0, rows_per_tile // zrows)
        def _(bz):
            pltpu.sync_copy(
                zbuf_v, acc_s.at[pl.ds(sid * rows_per_tile + bz * zrows, zrows)]
            )

        plsc.subcore_barrier()

        steps = (n_groups + NS - 1) // NS

        @pl.loop(0, steps)
        def _(i):
            b = sid + i * NS

            @pl.when(b < n_groups)
            def _():
                ebase = b * BK
                pltpu.sync_copy(m_h.at[pl.ds(ebase, BK)], vals_v)
                pltpu.sync_copy(
                    r_h.at[cid, pl.ds(b * GCHUNKS, GCHUNKS)], idx_v)
                for j in range(GCHUNKS):
                    pltpu.sync_copy(
                        vals_v.at[pl.ds(j * CHUNK, CHUNK)],
                        acc_s.at[idx_v.at[j]],
                        add=True,
                    )

        plsc.subcore_barrier()

        @pl.loop(0, rows_per_tile // zrows)
        def _(bz):
            base = sid * rows_per_tile + bz * zrows
            pltpu.sync_copy(
                acc_s.at[pl.ds(base, zrows)], out_h.at[cid, pl.ds(base, zrows)]
            )

    return k(msgs, idx_rm)


def _edge_call(sfp, rfp, ed, W1lo, W1hi, W1e, b1, W2, W2e, b2, ln_s, ln_b,
               E, D, DE):
    BE = 4000
    grid = (E // BE,)
    D2 = 2 * D

    def body(sfp_r, rfp_r, ed_r, W1lo_r, W1hi_r, W1e_r, b1_r, W2_r, W2e_r,
             b2_r, ln_s_r, ln_b_r, msgs_r, enew_r):
        xi = jnp.concatenate([sfp_r[...], rfp_r[...]], axis=1)
        lo = lax.bitcast_convert_type(
            lax.shift_left(xi, 16), jnp.float32).astype(jnp.bfloat16)
        hi = lax.bitcast_convert_type(
            lax.bitwise_and(xi, jnp.int32(-65536)),
            jnp.float32).astype(jnp.bfloat16)
        edb = ed_r[...].astype(jnp.bfloat16)
        dot = functools.partial(jnp.dot, preferred_element_type=jnp.float32)
        x1 = (dot(lo, W1lo_r[...]) + dot(hi, W1hi_r[...])
              + dot(edb, W1e_r[...]) + b1_r[...])
        h = jnp.maximum(x1, 0.0).astype(jnp.bfloat16)
        o2 = dot(h, W2_r[...]) + dot(edb, W2e_r[...]) + b2_r[...]
        msgs_r[...] = o2[:, :D]
        res = o2[:, D:]
        mu = jnp.mean(res, axis=-1, keepdims=True)
        var = jnp.mean((res - mu) ** 2, axis=-1, keepdims=True)
        enew_r[...] = ((res - mu) * lax.rsqrt(var + 1e-6) * ln_s_r[...]
                       + ln_b_r[...])

    full = lambda shape: pl.BlockSpec(shape, lambda i: (0,) * len(shape))
    return pl.pallas_call(
        body,
        grid=grid,
        in_specs=[
            pl.BlockSpec((BE, D // 2), lambda i: (i, 0)),
            pl.BlockSpec((BE, D // 2), lambda i: (i, 0)),
            pl.BlockSpec((BE, DE), lambda i: (i, 0)),
            full((D, D2)), full((D, D2)), full((DE, D2)), full((1, D2)),
            full((D2, D2)), full((DE, D2)), full((1, D2)),
            full((1, D)), full((1, D)),
        ],
        out_specs=[
            pl.BlockSpec((BE, D), lambda i: (i, 0)),
            pl.BlockSpec((BE, D), lambda i: (i, 0)),
        ],
        out_shape=[
            jax.ShapeDtypeStruct((E, D), jnp.float32),
            jax.ShapeDtypeStruct((E, D), jnp.float32),
        ],
    )(sfp, rfp, ed, W1lo, W1hi, W1e, b1, W2, W2e, b2, ln_s, ln_b)


def _node_call(nodes, p0, Wn1n, Wn1a, bn1, Wn2, bn2, W_node, ln_s, ln_b,
               N, D, H):
    BN = 2000
    grid = (N // BN,)

    def body(x_r, p0_r, Wn1n_r, Wn1a_r, bn1_r, Wn2_r, bn2_r, W_node_r,
             ln_s_r, ln_b_r, out_r):
        x = x_r[...].astype(jnp.bfloat16)
        agg = p0_r[...].astype(jnp.bfloat16)
        dot = functools.partial(jnp.dot, preferred_element_type=jnp.float32)
        h = jnp.maximum(dot(x, Wn1n_r[...]) + dot(agg, Wn1a_r[...])
                        + bn1_r[...], 0.0).astype(jnp.bfloat16)
        res = dot(x, W_node_r[...]) + dot(h, Wn2_r[...]) + bn2_r[...]
        mu = jnp.mean(res, axis=-1, keepdims=True)
        var = jnp.mean((res - mu) ** 2, axis=-1, keepdims=True)
        out_r[...] = ((res - mu) * lax.rsqrt(var + 1e-6) * ln_s_r[...]
                      + ln_b_r[...])

    full = lambda shape: pl.BlockSpec(shape, lambda i: (0,) * len(shape))
    return pl.pallas_call(
        body,
        grid=grid,
        in_specs=[
            pl.BlockSpec((BN, D), lambda i: (i, 0)),
            pl.BlockSpec((BN, D), lambda i: (i, 0)),
            full((D, H)), full((D, H)), full((1, H)),
            full((H, D)), full((1, D)), full((D, D)),
            full((1, D)), full((1, D)),
        ],
        out_specs=[pl.BlockSpec((BN, D), lambda i: (i, 0))],
        out_shape=[jax.ShapeDtypeStruct((N, D), jnp.float32)],
    )(nodes, p0, Wn1n, Wn1a, bn1, Wn2, bn2, W_node, ln_s, ln_b)[0]


def kernel(nodes, edges, senders, receivers, W_node, W_edge, Wm1, bm1, Wm2,
           bm2, Wn1, bn1, Wn2, bn2, We1, be1, We2, be2, ln_scale, ln_bias):
    N, D = nodes.shape
    E, DE = edges.shape
    H = Wm1.shape[1]

    s2d = senders.reshape(E // CHUNK, CHUNK)
    r2d = receivers.reshape(E // CHUNK, CHUNK)

    bf = jnp.bfloat16
    W1 = jnp.concatenate([
        jnp.concatenate([Wm1[:D], We1[:D]], axis=1),
        jnp.concatenate([Wm1[D:2 * D], We1[D:2 * D]], axis=1)], axis=0)
    W1e = jnp.concatenate([Wm1[2 * D:], We1[2 * D:]], axis=1)
    b1 = jnp.concatenate([bm1, be1]).reshape(1, 2 * H)
    zz = jnp.zeros((H, D), jnp.float32)
    W2 = jnp.concatenate([
        jnp.concatenate([Wm2, zz], axis=1),
        jnp.concatenate([zz, We2], axis=1)], axis=0)
    W2e = jnp.concatenate([jnp.zeros((DE, D), jnp.float32), W_edge], axis=1)
    b2 = jnp.concatenate([bm2, be2]).reshape(1, 2 * D)
    Wn1n, Wn1a = Wn1[:D].astype(bf), Wn1[D:].astype(bf)
    bn1_, bn2_ = bn1.reshape(1, H), bn2.reshape(1, D)
    ln_s, ln_b = ln_scale.reshape(1, D), ln_bias.reshape(1, D)

    nodes_pk = lax.bitcast_convert_type(
        nodes.astype(bf).reshape(N, D // 2, 2), jnp.int32)
    sfp, rfp = _gather_call(nodes_pk, s2d, r2d, E, D // 2)
    perm_lo = jnp.concatenate(
        [jnp.arange(0, D, 2), D + jnp.arange(0, D, 2)])
    W1lo = W1[perm_lo].astype(bf)
    W1hi = W1[perm_lo + 1].astype(bf)
    msgs, edges_new = _edge_call(
        sfp, rfp, edges, W1lo, W1hi, W1e.astype(bf), b1, W2.astype(bf),
        W2e.astype(bf), b2, ln_s, ln_b, E, D, DE)
    idx_rm = _remap_call(receivers.reshape(E // 512, 512))
    idx_rm = idx_rm.reshape(NC, E // CHUNK, CHUNK)
    partials = _scatter_call(msgs, idx_rm, E, D)
    agg = jnp.concatenate([partials[0, :NPH], partials[1, :N - NPH]], axis=0)
    nodes_new = _node_call(nodes, agg, Wn1n, Wn1a, bn1_, Wn2.astype(bf),
                           bn2_, W_node.astype(bf), ln_s, ln_b, N, D, H)
    return nodes_new, edges_new


# speedup vs baseline: 1.3158x; 1.3158x over previous
import functools

import jax
import jax.numpy as jnp
from jax import lax
from jax.experimental import pallas as pl
from jax.experimental.pallas import tpu as pltpu
from jax.experimental.pallas import tpu_sc as plsc

NC = 2
NS = 16
NW = NC * NS
LANES = 16
CHUNK = 128
GCHUNKS = 4
BK = CHUNK * GCHUNKS


def _sc_mesh(num_cores=NC):
    return plsc.VectorSubcoreMesh(core_axis_name="c", subcore_axis_name="s",
                                  num_cores=num_cores, num_subcores=NS)


def _gather_call(nodes, s2d, r2d, E, D):
    n_groups = E // BK
    steps = (n_groups + NW - 1) // NW
    dt = nodes.dtype

    @functools.partial(
        pl.kernel,
        out_type=jax.ShapeDtypeStruct((E, 2 * D), dt),
        mesh=_sc_mesh(),
        scratch_types=[
            pltpu.VMEM((GCHUNKS, CHUNK), jnp.int32),
            pltpu.VMEM((BK, D), dt),
            pltpu.SemaphoreType.DMA,
        ],
    )
    def k(nodes_h, s_h, r_h, sfrf_h, idx_v, rows_v, sem):
        wid = lax.axis_index("s") * NC + lax.axis_index("c")

        @pl.loop(0, steps)
        def _(i):
            b = wid + i * NW

            @pl.when(b < n_groups)
            def _():
                ebase = b * BK
                rbase = b * GCHUNKS

                def do(idx_src, col):
                    pltpu.sync_copy(idx_src.at[pl.ds(rbase, GCHUNKS)], idx_v)
                    cps = [
                        pltpu.async_copy(
                            nodes_h.at[idx_v.at[j]],
                            rows_v.at[pl.ds(j * CHUNK, CHUNK)],
                            sem,
                        )
                        for j in range(GCHUNKS)
                    ]
                    for c in cps:
                        c.wait()
                    pltpu.sync_copy(
                        rows_v, sfrf_h.at[pl.ds(ebase, BK), pl.ds(col, D)])

                do(s_h, 0)
                do(r_h, D)

    return k(nodes, s2d, r2d)


NPH = 5120
DUMP = 512
ACC_R = NPH + DUMP


def _remap_call(r_rs):
    R0, R1 = r_rs.shape

    def body(r_ref, o_ref):
        idx = r_ref[...]
        for c in range(NC):
            rel = idx - c * NPH
            inr = (rel >= 0) & (rel < NPH)
            o_ref[c, :, :] = jnp.where(inr, rel, NPH + (idx & (DUMP - 1)))

    return pl.pallas_call(
        body,
        out_shape=jax.ShapeDtypeStruct((NC, R0, R1), jnp.int32),
    )(r_rs)


def _scatter_call(msgs, idx_rm, g0, g1, D):
    rows_per_tile = ACC_R // NS
    zrows = 32
    assert rows_per_tile % zrows == 0
    n_range = g1 - g0

    @functools.partial(
        pl.kernel,
        out_type=jax.ShapeDtypeStruct((NC, ACC_R, D), jnp.float32),
        mesh=_sc_mesh(),
        scratch_types=[
            pltpu.VMEM((GCHUNKS, CHUNK), jnp.int32),
            pltpu.VMEM((BK, D), jnp.float32),
            pltpu.VMEM((zrows, D), jnp.float32),
            pltpu.VMEM_SHARED((ACC_R, D), jnp.float32),
        ],
    )
    def k(m_h, r_h, out_h, idx_v, vals_v, zbuf_v, acc_s):
        cid = lax.axis_index("c")
        sid = lax.axis_index("s")

        @pl.loop(0, zrows)
        def _(r):
            @pl.loop(0, D, step=LANES)
            def _(c2):
                zbuf_v[r, pl.ds(c2, LANES)] = jnp.zeros((LANES,), jnp.float32)

        @pl.loop(0, rows_per_tile // zrows)
        def _(bz):
            pltpu.sync_copy(
                zbuf_v, acc_s.at[pl.ds(sid * rows_per_tile + bz * zrows, zrows)]
            )

        plsc.subcore_barrier()

        steps = (n_range + NS - 1) // NS

        @pl.loop(0, steps)
        def _(i):
            b = g0 + sid + i * NS

            @pl.when(b < g1)
            def _():
                pltpu.sync_copy(m_h.at[pl.ds((b - g0) * BK, BK)], vals_v)
                pltpu.sync_copy(
                    r_h.at[cid, pl.ds(b * GCHUNKS, GCHUNKS)], idx_v)
                for j in range(GCHUNKS):
                    pltpu.sync_copy(
                        vals_v.at[pl.ds(j * CHUNK, CHUNK)],
                        acc_s.at[idx_v.at[j]],
                        add=True,
                    )

        plsc.subcore_barrier()

        @pl.loop(0, rows_per_tile // zrows)
        def _(bz):
            base = sid * rows_per_tile + bz * zrows
            pltpu.sync_copy(
                acc_s.at[pl.ds(base, zrows)], out_h.at[cid, pl.ds(base, zrows)]
            )

    return k(msgs, idx_rm)


def _edge_call(sfrf, ed, W1, W1e, b1, W2, W2e, b2, ln_s, ln_b,
               E_c, off_e, E, D, DE, prev=None):
    BE = 4000
    grid = (E_c // BE,)
    off = off_e // BE
    D2 = 2 * D

    def body(x_r, ed_r, W1_r, W1e_r, b1_r, W2_r, W2e_r, b2_r, ln_s_r,
             ln_b_r, *rest):
        msgs_r, enew_r = rest[-2], rest[-1]
        xb = x_r[...].astype(jnp.bfloat16)
        edb = ed_r[...].astype(jnp.bfloat16)
        dot = functools.partial(jnp.dot, preferred_element_type=jnp.float32)
        x1 = dot(xb, W1_r[...]) + dot(edb, W1e_r[...]) + b1_r[...]
        h = jnp.maximum(x1, 0.0).astype(jnp.bfloat16)
        o2 = dot(h, W2_r[...]) + dot(edb, W2e_r[...]) + b2_r[...]
        msgs_r[...] = o2[:, :D]
        res = o2[:, D:]
        mu = jnp.mean(res, axis=-1, keepdims=True)
        var = jnp.mean((res - mu) ** 2, axis=-1, keepdims=True)
        enew_r[...] = ((res - mu) * lax.rsqrt(var + 1e-6) * ln_s_r[...]
                       + ln_b_r[...])

    full = lambda shape: pl.BlockSpec(shape, lambda i: (0,) * len(shape))
    in_specs = [
        pl.BlockSpec((BE, D2), lambda i: (i, 0)),
        pl.BlockSpec((BE, DE), lambda i: (i + off, 0)),
        full((D2, D2)), full((DE, D2)), full((1, D2)),
        full((D2, D2)), full((DE, D2)), full((1, D2)),
        full((1, D)), full((1, D)),
    ]
    args = [sfrf, ed, W1, W1e, b1, W2, W2e, b2, ln_s, ln_b]
    aliases = {}
    if prev is not None:
        in_specs += [pl.BlockSpec(memory_space=pl.ANY)]
        args += [prev]
        aliases = {len(args) - 1: 1}
    return pl.pallas_call(
        body,
        grid=grid,
        in_specs=in_specs,
        out_specs=[
            pl.BlockSpec((BE, D), lambda i: (i, 0)),
            pl.BlockSpec((BE, D), lambda i: (i + off, 0)),
        ],
        out_shape=[
            jax.ShapeDtypeStruct((E_c, D), jnp.float32),
            jax.ShapeDtypeStruct((E, D), jnp.float32),
        ],
        input_output_aliases=aliases,
    )(*args)


def _node_call(nodes, p0, p1, Wn1n, Wn1a, bn1, Wn2, bn2, W_node, ln_s, ln_b,
               N, D, H):
    BN = 2000
    grid = (N // BN,)

    def body(x_r, p0_r, p1_r, Wn1n_r, Wn1a_r, bn1_r, Wn2_r, bn2_r, W_node_r,
             ln_s_r, ln_b_r, out_r):
        x = x_r[...].astype(jnp.bfloat16)
        agg = (p0_r[...] + p1_r[...]).astype(jnp.bfloat16)
        dot = functools.partial(jnp.dot, preferred_element_type=jnp.float32)
        h = jnp.maximum(dot(x, Wn1n_r[...]) + dot(agg, Wn1a_r[...])
                        + bn1_r[...], 0.0).astype(jnp.bfloat16)
        res = dot(x, W_node_r[...]) + dot(h, Wn2_r[...]) + bn2_r[...]
        mu = jnp.mean(res, axis=-1, keepdims=True)
        var = jnp.mean((res - mu) ** 2, axis=-1, keepdims=True)
        out_r[...] = ((res - mu) * lax.rsqrt(var + 1e-6) * ln_s_r[...]
                      + ln_b_r[...])

    full = lambda shape: pl.BlockSpec(shape, lambda i: (0,) * len(shape))
    return pl.pallas_call(
        body,
        grid=grid,
        in_specs=[
            pl.BlockSpec((BN, D), lambda i: (i, 0)),
            pl.BlockSpec((BN, D), lambda i: (i, 0)),
            pl.BlockSpec((BN, D), lambda i: (i, 0)),
            full((D, H)), full((D, H)), full((1, H)),
            full((H, D)), full((1, D)), full((D, D)),
            full((1, D)), full((1, D)),
        ],
        out_specs=[pl.BlockSpec((BN, D), lambda i: (i, 0))],
        out_shape=[jax.ShapeDtypeStruct((N, D), jnp.float32)],
    )(nodes, p0, p1, Wn1n, Wn1a, bn1, Wn2, bn2, W_node, ln_s, ln_b)[0]


def kernel(nodes, edges, senders, receivers, W_node, W_edge, Wm1, bm1, Wm2,
           bm2, Wn1, bn1, Wn2, bn2, We1, be1, We2, be2, ln_scale, ln_bias):
    N, D = nodes.shape
    E, DE = edges.shape
    H = Wm1.shape[1]

    s2d = senders.reshape(E // CHUNK, CHUNK)
    r2d = receivers.reshape(E // CHUNK, CHUNK)

    bf = jnp.bfloat16
    W1 = jnp.concatenate([
        jnp.concatenate([Wm1[:D], We1[:D]], axis=1),
        jnp.concatenate([Wm1[D:2 * D], We1[D:2 * D]], axis=1)], axis=0)
    W1e = jnp.concatenate([Wm1[2 * D:], We1[2 * D:]], axis=1)
    b1 = jnp.concatenate([bm1, be1]).reshape(1, 2 * H)
    zz = jnp.zeros((H, D), jnp.float32)
    W2 = jnp.concatenate([
        jnp.concatenate([Wm2, zz], axis=1),
        jnp.concatenate([zz, We2], axis=1)], axis=0)
    W2e = jnp.concatenate([jnp.zeros((DE, D), jnp.float32), W_edge], axis=1)
    b2 = jnp.concatenate([bm2, be2]).reshape(1, 2 * D)
    Wn1n, Wn1a = Wn1[:D].astype(bf), Wn1[D:].astype(bf)
    bn1_, bn2_ = bn1.reshape(1, H), bn2.reshape(1, D)
    ln_s, ln_b = ln_scale.reshape(1, D), ln_bias.reshape(1, D)

    idx_rm = _remap_call(receivers.reshape(E // 512, 512))
    idx_rm = idx_rm.reshape(NC, E // CHUNK, CHUNK)

    C1 = 192000
    C2 = E - C1
    Wb = (W1.astype(bf), W1e.astype(bf), b1, W2.astype(bf), W2e.astype(bf),
          b2, ln_s, ln_b)
    sfrf1 = _gather_call(nodes, s2d[:C1 // CHUNK], r2d[:C1 // CHUNK], C1, D)
    sfrf2 = _gather_call(nodes, s2d[C1 // CHUNK:], r2d[C1 // CHUNK:], C2, D)
    msgs, edges_new = _edge_call(sfrf1, edges, *Wb, C1, 0, E, D, DE)
    p1 = _scatter_call(msgs, idx_rm, 0, C1 // BK, D)
    msgs, edges_new = _edge_call(sfrf2, edges, *Wb, C2, C1, E, D, DE,
                                 prev=edges_new)
    p2 = _scatter_call(msgs, idx_rm, C1 // BK, E // BK, D)
    agg1 = jnp.concatenate([p1[0, :NPH], p1[1, :N - NPH]], axis=0)
    agg2 = jnp.concatenate([p2[0, :NPH], p2[1, :N - NPH]], axis=0)
    nodes_new = _node_call(nodes, agg1, agg2, Wn1n, Wn1a, bn1_,
                           Wn2.astype(bf), bn2_, W_node.astype(bf),
                           ln_s, ln_b, N, D, H)
    return nodes_new, edges_new


# speedup vs baseline: 1.4443x; 1.0977x over previous
import functools

import jax
import jax.numpy as jnp
from jax import lax
from jax.experimental import pallas as pl
from jax.experimental.pallas import tpu as pltpu
from jax.experimental.pallas import tpu_sc as plsc

NC = 2
NS = 16
NW = NC * NS
LANES = 16
CHUNK = 128
GCHUNKS = 4
BK = CHUNK * GCHUNKS


def _sc_mesh(num_cores=NC):
    return plsc.VectorSubcoreMesh(core_axis_name="c", subcore_axis_name="s",
                                  num_cores=num_cores, num_subcores=NS)


def _gather_call(nodes, s2d, r2d, E, D):
    GB = CHUNK
    n_groups = E // GB
    steps = (n_groups + NW - 1) // NW
    steps2 = (steps + 1) // 2
    dt = nodes.dtype

    @functools.partial(
        pl.kernel,
        out_type=jax.ShapeDtypeStruct((E, 2 * D), dt),
        mesh=_sc_mesh(),
        scratch_types=[
            pltpu.VMEM((2, CHUNK), jnp.int32),
            pltpu.VMEM((2, CHUNK), jnp.int32),
            pltpu.VMEM((GB, D), dt),
            pltpu.VMEM((GB, D), dt),
            pltpu.VMEM((GB, D), dt),
            pltpu.VMEM((GB, D), dt),
            pltpu.SemaphoreType.DMA,
            pltpu.SemaphoreType.DMA,
            pltpu.SemaphoreType.DMA,
        ],
    )
    def k(nodes_h, s_h, r_h, sfrf_h, idxs_v, idxr_v, rs0, rs1, rr0, rr1,
          sem_g, sem_w0, sem_w1):
        wid = lax.axis_index("s") * NC + lax.axis_index("c")
        bufs = ((rs0, rr0, sem_w0), (rs1, rr1, sem_w1))

        @pl.loop(0, steps2)
        def _(o):
            for par in range(2):
                rows_s, rows_r, sem_w = bufs[par]
                i = 2 * o + par
                b = wid + i * NW

                @pl.when(b < n_groups)
                def _():
                    ebase = b * GB

                    @pl.when(o > 0)
                    def _():
                        pltpu.make_async_copy(
                            rows_s, sfrf_h.at[pl.ds(0, GB), pl.ds(0, D)],
                            sem_w).wait()
                        pltpu.make_async_copy(
                            rows_r, sfrf_h.at[pl.ds(0, GB), pl.ds(D, D)],
                            sem_w).wait()

                    pltpu.sync_copy(s_h.at[pl.ds(b, 1)],
                                    idxs_v.at[pl.ds(par, 1)])
                    pltpu.sync_copy(r_h.at[pl.ds(b, 1)],
                                    idxr_v.at[pl.ds(par, 1)])
                    g1 = pltpu.async_copy(
                        nodes_h.at[idxs_v.at[par]], rows_s, sem_g)
                    g2 = pltpu.async_copy(
                        nodes_h.at[idxr_v.at[par]], rows_r, sem_g)
                    g1.wait()
                    g2.wait()
                    pltpu.async_copy(
                        rows_s, sfrf_h.at[pl.ds(ebase, GB), pl.ds(0, D)],
                        sem_w)
                    pltpu.async_copy(
                        rows_r, sfrf_h.at[pl.ds(ebase, GB), pl.ds(D, D)],
                        sem_w)

        for par in range(2):
            rows_s, rows_r, sem_w = bufs[par]
            pltpu.make_async_copy(
                rows_s, sfrf_h.at[pl.ds(0, GB), pl.ds(0, D)], sem_w).wait()
            pltpu.make_async_copy(
                rows_r, sfrf_h.at[pl.ds(0, GB), pl.ds(D, D)], sem_w).wait()

    return k(nodes, s2d, r2d)


NPH = 5120
DUMP = 512
ACC_R = NPH + DUMP


def _remap_call(r_rs):
    R0, R1 = r_rs.shape

    def body(r_ref, o_ref):
        idx = r_ref[...]
        for c in range(NC):
            rel = idx - c * NPH
            inr = (rel >= 0) & (rel < NPH)
            o_ref[c, :, :] = jnp.where(inr, rel, NPH + (idx & (DUMP - 1)))

    return pl.pallas_call(
        body,
        out_shape=jax.ShapeDtypeStruct((NC, R0, R1), jnp.int32),
    )(r_rs)


def _scatter_call(msgs, idx_rm, g0, g1, D):
    rows_per_tile = ACC_R // NS
    zrows = 32
    assert rows_per_tile % zrows == 0
    n_range = g1 - g0

    @functools.partial(
        pl.kernel,
        out_type=jax.ShapeDtypeStruct((NC, ACC_R, D), jnp.float32),
        mesh=_sc_mesh(),
        scratch_types=[
            pltpu.VMEM((GCHUNKS, CHUNK), jnp.int32),
            pltpu.VMEM((BK, D), jnp.float32),
            pltpu.VMEM((zrows, D), jnp.float32),
            pltpu.VMEM_SHARED((ACC_R, D), jnp.float32),
        ],
    )
    def k(m_h, r_h, out_h, idx_v, vals_v, zbuf_v, acc_s):
        cid = lax.axis_index("c")
        sid = lax.axis_index("s")

        @pl.loop(0, zrows)
        def _(r):
            @pl.loop(0, D, step=LANES)
            def _(c2):
                zbuf_v[r, pl.ds(c2, LANES)] = jnp.zeros((LANES,), jnp.float32)

        @pl.loop(0, rows_per_tile // zrows)
        def _(bz):
            pltpu.sync_copy(
                zbuf_v, acc_s.at[pl.ds(sid * rows_per_tile + bz * zrows, zrows)]
            )

        plsc.subcore_barrier()

        steps = (n_range + NS - 1) // NS

        @pl.loop(0, steps)
        def _(i):
            b = g0 + sid + i * NS

            @pl.when(b < g1)
            def _():
                pltpu.sync_copy(m_h.at[pl.ds((b - g0) * BK, BK)], vals_v)
                pltpu.sync_copy(
                    r_h.at[cid, pl.ds(b * GCHUNKS, GCHUNKS)], idx_v)
                for j in range(GCHUNKS):
                    pltpu.sync_copy(
                        vals_v.at[pl.ds(j * CHUNK, CHUNK)],
                        acc_s.at[idx_v.at[j]],
                        add=True,
                    )

        plsc.subcore_barrier()

        @pl.loop(0, rows_per_tile // zrows)
        def _(bz):
            base = sid * rows_per_tile + bz * zrows
            pltpu.sync_copy(
                acc_s.at[pl.ds(base, zrows)], out_h.at[cid, pl.ds(base, zrows)]
            )

    return k(msgs, idx_rm)


def _edge_call(sfrf, ed, W1, W1e, b1, W2, W2e, b2, ln_s, ln_b,
               E_c, off_e, E, D, DE, prev=None):
    BE = 4000
    grid = (E_c // BE,)
    off = off_e // BE
    D2 = 2 * D

    def body(x_r, ed_r, W1_r, W1e_r, b1_r, W2_r, W2e_r, b2_r, ln_s_r,
             ln_b_r, *rest):
        msgs_r, enew_r = rest[-2], rest[-1]
        xb = x_r[...].astype(jnp.bfloat16)
        edb = ed_r[...].astype(jnp.bfloat16)
        dot = functools.partial(jnp.dot, preferred_element_type=jnp.float32)
        x1 = dot(xb, W1_r[...]) + dot(edb, W1e_r[...]) + b1_r[...]
        h = jnp.maximum(x1, 0.0).astype(jnp.bfloat16)
        o2 = dot(h, W2_r[...]) + dot(edb, W2e_r[...]) + b2_r[...]
        msgs_r[...] = o2[:, :D]
        res = o2[:, D:]
        mu = jnp.mean(res, axis=-1, keepdims=True)
        var = jnp.mean((res - mu) ** 2, axis=-1, keepdims=True)
        enew_r[...] = ((res - mu) * lax.rsqrt(var + 1e-6) * ln_s_r[...]
                       + ln_b_r[...])

    full = lambda shape: pl.BlockSpec(shape, lambda i: (0,) * len(shape))
    in_specs = [
        pl.BlockSpec((BE, D2), lambda i: (i, 0)),
        pl.BlockSpec((BE, DE), lambda i: (i + off, 0)),
        full((D2, D2)), full((DE, D2)), full((1, D2)),
        full((D2, D2)), full((DE, D2)), full((1, D2)),
        full((1, D)), full((1, D)),
    ]
    args = [sfrf, ed, W1, W1e, b1, W2, W2e, b2, ln_s, ln_b]
    aliases = {}
    if prev is not None:
        in_specs += [pl.BlockSpec(memory_space=pl.ANY)]
        args += [prev]
        aliases = {len(args) - 1: 1}
    return pl.pallas_call(
        body,
        grid=grid,
        in_specs=in_specs,
        out_specs=[
            pl.BlockSpec((BE, D), lambda i: (i, 0)),
            pl.BlockSpec((BE, D), lambda i: (i + off, 0)),
        ],
        out_shape=[
            jax.ShapeDtypeStruct((E_c, D), jnp.float32),
            jax.ShapeDtypeStruct((E, D), jnp.float32),
        ],
        input_output_aliases=aliases,
    )(*args)


def _node_call(nodes, p0, p1, Wn1n, Wn1a, bn1, Wn2, bn2, W_node, ln_s, ln_b,
               N, D, H):
    BN = 2000
    grid = (N // BN,)

    def body(x_r, p0_r, p1_r, Wn1n_r, Wn1a_r, bn1_r, Wn2_r, bn2_r, W_node_r,
             ln_s_r, ln_b_r, out_r):
        x = x_r[...].astype(jnp.bfloat16)
        agg = (p0_r[...] + p1_r[...]).astype(jnp.bfloat16)
        dot = functools.partial(jnp.dot, preferred_element_type=jnp.float32)
        h = jnp.maximum(dot(x, Wn1n_r[...]) + dot(agg, Wn1a_r[...])
                        + bn1_r[...], 0.0).astype(jnp.bfloat16)
        res = dot(x, W_node_r[...]) + dot(h, Wn2_r[...]) + bn2_r[...]
        mu = jnp.mean(res, axis=-1, keepdims=True)
        var = jnp.mean((res - mu) ** 2, axis=-1, keepdims=True)
        out_r[...] = ((res - mu) * lax.rsqrt(var + 1e-6) * ln_s_r[...]
                      + ln_b_r[...])

    full = lambda shape: pl.BlockSpec(shape, lambda i: (0,) * len(shape))
    return pl.pallas_call(
        body,
        grid=grid,
        in_specs=[
            pl.BlockSpec((BN, D), lambda i: (i, 0)),
            pl.BlockSpec((BN, D), lambda i: (i, 0)),
            pl.BlockSpec((BN, D), lambda i: (i, 0)),
            full((D, H)), full((D, H)), full((1, H)),
            full((H, D)), full((1, D)), full((D, D)),
            full((1, D)), full((1, D)),
        ],
        out_specs=[pl.BlockSpec((BN, D), lambda i: (i, 0))],
        out_shape=[jax.ShapeDtypeStruct((N, D), jnp.float32)],
    )(nodes, p0, p1, Wn1n, Wn1a, bn1, Wn2, bn2, W_node, ln_s, ln_b)[0]


def kernel(nodes, edges, senders, receivers, W_node, W_edge, Wm1, bm1, Wm2,
           bm2, Wn1, bn1, Wn2, bn2, We1, be1, We2, be2, ln_scale, ln_bias):
    N, D = nodes.shape
    E, DE = edges.shape
    H = Wm1.shape[1]

    s2d = senders.reshape(E // CHUNK, CHUNK)
    r2d = receivers.reshape(E // CHUNK, CHUNK)

    bf = jnp.bfloat16
    W1 = jnp.concatenate([
        jnp.concatenate([Wm1[:D], We1[:D]], axis=1),
        jnp.concatenate([Wm1[D:2 * D], We1[D:2 * D]], axis=1)], axis=0)
    W1e = jnp.concatenate([Wm1[2 * D:], We1[2 * D:]], axis=1)
    b1 = jnp.concatenate([bm1, be1]).reshape(1, 2 * H)
    zz = jnp.zeros((H, D), jnp.float32)
    W2 = jnp.concatenate([
        jnp.concatenate([Wm2, zz], axis=1),
        jnp.concatenate([zz, We2], axis=1)], axis=0)
    W2e = jnp.concatenate([jnp.zeros((DE, D), jnp.float32), W_edge], axis=1)
    b2 = jnp.concatenate([bm2, be2]).reshape(1, 2 * D)
    Wn1n, Wn1a = Wn1[:D].astype(bf), Wn1[D:].astype(bf)
    bn1_, bn2_ = bn1.reshape(1, H), bn2.reshape(1, D)
    ln_s, ln_b = ln_scale.reshape(1, D), ln_bias.reshape(1, D)

    idx_rm = _remap_call(receivers.reshape(E // 512, 512))
    idx_rm = idx_rm.reshape(NC, E // CHUNK, CHUNK)

    C1 = 192000
    C2 = E - C1
    Wb = (W1.astype(bf), W1e.astype(bf), b1, W2.astype(bf), W2e.astype(bf),
          b2, ln_s, ln_b)
    sfrf1 = _gather_call(nodes, s2d[:C1 // CHUNK], r2d[:C1 // CHUNK], C1, D)
    sfrf2 = _gather_call(nodes, s2d[C1 // CHUNK:], r2d[C1 // CHUNK:], C2, D)
    msgs, edges_new = _edge_call(sfrf1, edges, *Wb, C1, 0, E, D, DE)
    p1 = _scatter_call(msgs, idx_rm, 0, C1 // BK, D)
    msgs, edges_new = _edge_call(sfrf2, edges, *Wb, C2, C1, E, D, DE,
                                 prev=edges_new)
    p2 = _scatter_call(msgs, idx_rm, C1 // BK, E // BK, D)
    agg1 = jnp.concatenate([p1[0, :NPH], p1[1, :N - NPH]], axis=0)
    agg2 = jnp.concatenate([p2[0, :NPH], p2[1, :N - NPH]], axis=0)
    nodes_new = _node_call(nodes, agg1, agg2, Wn1n, Wn1a, bn1_,
                           Wn2.astype(bf), bn2_, W_node.astype(bf),
                           ln_s, ln_b, N, D, H)
    return nodes_new, edges_new
